# baseline (speedup 1.0000x reference)
import functools

import jax
import jax.numpy as jnp
from jax import lax
from jax.experimental import pallas as pl
from jax.experimental.pallas import tpu as pltpu
from jax.experimental.pallas import tpu_sc as plsc

NUM_BIOMES = 11
EMBED_DIM = 64
BATCH = 16384

_info = plsc.get_sparse_core_info()
_NC = _info.num_cores
_NS = _info.num_subcores
_NW = _NC * _NS
_BPW = BATCH // _NW
_L = 16
_CHUNKR = 128
_NCHUNK = _BPW // _CHUNKR

_mesh = plsc.VectorSubcoreMesh(core_axis_name="c", subcore_axis_name="s")


@functools.partial(
    pl.kernel,
    mesh=_mesh,
    out_type=jax.ShapeDtypeStruct((BATCH, EMBED_DIM), jnp.float32),
    scratch_types=[
        pltpu.VMEM((NUM_BIOMES * EMBED_DIM,), jnp.float32),
        pltpu.VMEM((_BPW,), jnp.int32),
        pltpu.VMEM((_BPW, EMBED_DIM), jnp.float32),
        pltpu.SemaphoreType.DMA,
    ],
    compiler_params=pltpu.CompilerParams(
        use_tc_tiling_on_sc=False, needs_layout_passes=False
    ),
)
def _emb_lookup(table_hbm, idx_hbm, out_hbm, table_v, idx_v, rows_v, sem):
    wid = lax.axis_index("s") * _NC + lax.axis_index("c")
    base = wid * _BPW
    pltpu.sync_copy(table_hbm, table_v)
    pltpu.sync_copy(idx_hbm.at[pl.ds(base, _BPW)], idx_v)

    chunk_groups = _CHUNKR // _L
    copies = []
    for c in range(_NCHUNK):

        @plsc.parallel_loop(c * chunk_groups, (c + 1) * chunk_groups, unroll=1)
        def _group(g):
            pvec = idx_v[pl.ds(g * _L, _L)] * EMBED_DIM
            for r in range(_L):
                src = pvec[r]
                row = g * _L + r
                for k in range(EMBED_DIM // _L):
                    rows_v[row, pl.ds(k * _L, _L)] = table_v[pl.ds(src + k * _L, _L)]

        cp = pltpu.make_async_copy(
            rows_v.at[pl.ds(c * _CHUNKR, _CHUNKR)],
            out_hbm.at[pl.ds(base + c * _CHUNKR, _CHUNKR)],
            sem,
        )
        cp.start()
        copies.append(cp)

    for cp in copies:
        cp.wait()


def kernel(prompts, table):
    return _emb_lookup(table.reshape(-1), prompts.astype(jnp.int32))


# speedup vs baseline: 1.1864x; 1.1864x over previous
import functools

import jax
import jax.numpy as jnp
from jax import lax
from jax.experimental import pallas as pl
from jax.experimental.pallas import tpu as pltpu
from jax.experimental.pallas import tpu_sc as plsc

NUM_BIOMES = 11
EMBED_DIM = 64
BATCH = 16384

_info = plsc.get_sparse_core_info()
_NC = _info.num_cores
_NS = _info.num_subcores
_NW = _NC * _NS
_BPW = BATCH // _NW
_L = 16
_GROUPS = _BPW // _L

_mesh = plsc.VectorSubcoreMesh(core_axis_name="c", subcore_axis_name="s")


@functools.partial(
    pl.kernel,
    mesh=_mesh,
    out_type=jax.ShapeDtypeStruct((EMBED_DIM, BATCH), jnp.float32),
    scratch_types=[
        pltpu.VMEM((NUM_BIOMES * EMBED_DIM,), jnp.float32),
        pltpu.VMEM((_BPW,), jnp.int32),
        pltpu.VMEM((EMBED_DIM, _BPW), jnp.float32),
        pltpu.SemaphoreType.DMA,
    ],
    compiler_params=pltpu.CompilerParams(needs_layout_passes=False),
)
def _emb_lookup(table_hbm, idx_hbm, out_hbm, table_v, idx_v, rows_v, sem):
    wid = lax.axis_index("s") * _NC + lax.axis_index("c")
    base = wid * _BPW
    pltpu.sync_copy(table_hbm, table_v)
    pltpu.sync_copy(idx_hbm.at[pl.ds(base, _BPW)], idx_v)

    @plsc.parallel_loop(0, _GROUPS, unroll=1)
    def _group(g):
        pvec = idx_v[pl.ds(g * _L, _L)] * EMBED_DIM
        c0 = g * _L
        for j in range(EMBED_DIM):
            rows_v[j, pl.ds(c0, _L)] = plsc.load_gather(table_v, [pvec + j])

    pltpu.async_copy(rows_v, out_hbm.at[:, pl.ds(base, _BPW)], sem).wait()


def kernel(prompts, table):
    out_t = _emb_lookup(table.reshape(-1), prompts.astype(jnp.int32))
    return out_t.T


# speedup vs baseline: 1.2174x; 1.0261x over previous
import functools

import jax
import jax.numpy as jnp
from jax import lax
from jax.experimental import pallas as pl
from jax.experimental.pallas import tpu as pltpu
from jax.experimental.pallas import tpu_sc as plsc

NUM_BIOMES = 11
EMBED_DIM = 64
BATCH = 16384

_info = plsc.get_sparse_core_info()
_NC = _info.num_cores
_NS = _info.num_subcores
_NW = _NC * _NS
_BPW = BATCH // _NW
_L = 16
_GROUPS = _BPW // _L
_CHUNKG = 8
_NCHUNK = _GROUPS // _CHUNKG
_DEPTH = 6

_mesh = plsc.VectorSubcoreMesh(core_axis_name="c", subcore_axis_name="s")


@functools.partial(
    pl.kernel,
    mesh=_mesh,
    out_type=jax.ShapeDtypeStruct((EMBED_DIM, BATCH), jnp.float32),
    scratch_types=[
        pltpu.VMEM((NUM_BIOMES * EMBED_DIM,), jnp.float32),
        pltpu.VMEM((_BPW,), jnp.int32),
        pltpu.VMEM((EMBED_DIM, _BPW), jnp.float32),
        pltpu.SemaphoreType.DMA,
    ],
    compiler_params=pltpu.CompilerParams(needs_layout_passes=False),
)
def _emb_lookup(table_hbm, idx_hbm, out_hbm, table_v, idx_v, rows_v, sem):
    wid = lax.axis_index("s") * _NC + lax.axis_index("c")
    base = wid * _BPW
    pltpu.sync_copy(table_hbm, table_v)
    pltpu.sync_copy(idx_hbm.at[pl.ds(base, _BPW)], idx_v)

    copies = []
    for c in range(_NCHUNK):

        @plsc.parallel_loop(c * _CHUNKG, (c + 1) * _CHUNKG, unroll=1)
        def _group(g):
            pvec = idx_v[pl.ds(g * _L, _L)] * EMBED_DIM
            c0 = g * _L
            vals = {}
            for j in range(EMBED_DIM + _DEPTH):
                if j < EMBED_DIM:
                    vals[j] = plsc.load_gather(table_v, [pvec + j])
                if j >= _DEPTH:
                    rows_v[j - _DEPTH, pl.ds(c0, _L)] = vals.pop(j - _DEPTH)

        cp = pltpu.make_async_copy(
            rows_v.at[:, pl.ds(c * _CHUNKG * _L, _CHUNKG * _L)],
            out_hbm.at[:, pl.ds(base + c * _CHUNKG * _L, _CHUNKG * _L)],
            sem,
        )
        cp.start()
        copies.append(cp)

    for cp in copies:
        cp.wait()


def kernel(prompts, table):
    out_t = _emb_lookup(table.reshape(-1), prompts.astype(jnp.int32))
    return out_t.T


# speedup vs baseline: 1.5264x; 1.2539x over previous
import functools

import jax
import jax.numpy as jnp
from jax import lax
from jax.experimental import pallas as pl
from jax.experimental.pallas import tpu as pltpu
from jax.experimental.pallas import tpu_sc as plsc

NUM_BIOMES = 11
EMBED_DIM = 64
BATCH = 16384

_info = plsc.get_sparse_core_info()
_NC = _info.num_cores
_NS = _info.num_subcores
_NW = _NC * _NS
_BPW = BATCH // _NW
_L = 16
_GROUPS = _BPW // _L
_CHUNKG = 8
_NCHUNK = _GROUPS // _CHUNKG
_DEPTH = 6
_TSIZE = NUM_BIOMES * EMBED_DIM
_TSTRIDE = _TSIZE + 1

_mesh = plsc.VectorSubcoreMesh(core_axis_name="c", subcore_axis_name="s")


@functools.partial(
    pl.kernel,
    mesh=_mesh,
    out_type=jax.ShapeDtypeStruct((EMBED_DIM, BATCH), jnp.float32),
    scratch_types=[
        pltpu.VMEM((_L * _TSTRIDE,), jnp.float32),
        pltpu.VMEM((_BPW,), jnp.int32),
        pltpu.VMEM((EMBED_DIM, _BPW), jnp.float32),
        pltpu.SemaphoreType.DMA,
        pltpu.SemaphoreType.DMA,
    ],
    compiler_params=pltpu.CompilerParams(needs_layout_passes=False),
)
def _emb_lookup(table_hbm, idx_hbm, out_hbm, table_v, idx_v, rows_v, sem, tsem):
    wid = lax.axis_index("s") * _NC + lax.axis_index("c")
    base = wid * _BPW

    tcp = pltpu.make_async_copy(table_hbm, table_v, tsem)
    tcp.start()
    pltpu.sync_copy(idx_hbm.at[pl.ds(base, _BPW)], idx_v)
    tcp.wait()

    lane_off = lax.iota(jnp.int32, _L) * _TSTRIDE

    copies = []
    for c in range(_NCHUNK):

        @plsc.parallel_loop(c * _CHUNKG, (c + 1) * _CHUNKG, unroll=1)
        def _group(g):
            pvec = idx_v[pl.ds(g * _L, _L)] * EMBED_DIM + lane_off
            c0 = g * _L
            vals = {}
            for j in range(EMBED_DIM + _DEPTH):
                if j < EMBED_DIM:
                    vals[j] = plsc.load_gather(table_v, [pvec + j])
                if j >= _DEPTH:
                    rows_v[j - _DEPTH, pl.ds(c0, _L)] = vals.pop(j - _DEPTH)

        cp = pltpu.make_async_copy(
            rows_v.at[:, pl.ds(c * _CHUNKG * _L, _CHUNKG * _L)],
            out_hbm.at[:, pl.ds(base + c * _CHUNKG * _L, _CHUNKG * _L)],
            sem,
        )
        cp.start()
        copies.append(cp)

    for cp in copies:
        cp.wait()


def kernel(prompts, table):
    rep = jnp.tile(jnp.pad(table.reshape(-1), (0, 1)), _L)
    out_t = _emb_lookup(rep, prompts.astype(jnp.int32))
    return out_t.T
